# Initial kernel scaffold; baseline (speedup 1.0000x reference)
import jax
import jax.numpy as jnp
from jax.experimental import pallas as pl


def kernel(input_ids, attention_mask, position_ids, labels, weight):
    raise NotImplementedError("write your pallas kernel here")


# speedup vs baseline: 1.4808x; 1.4808x over previous
import functools

import jax
import jax.numpy as jnp
from jax import lax
from jax.experimental import pallas as pl
from jax.experimental.pallas import tpu as pltpu
from jax.experimental.pallas import tpu_sc as plsc

INT_MIN = jnp.iinfo(jnp.int32).min


@functools.lru_cache(maxsize=None)
def _make_gather(V, D, N):
    info = plsc.get_sparse_core_info()
    NC, NS = info.num_cores, info.num_subcores
    NW = NC * NS
    assert N % NW == 0
    n_per_w = N // NW
    CH = 64
    assert n_per_w % CH == 0
    n_ch = n_per_w // CH
    mesh = plsc.VectorSubcoreMesh(core_axis_name="c", subcore_axis_name="s")

    @functools.partial(
        pl.kernel,
        mesh=mesh,
        out_type=jax.ShapeDtypeStruct((N, D), jnp.float32),
        scratch_types=[
            pltpu.VMEM((n_ch, CH), jnp.int32),
            pltpu.VMEM((CH, D), jnp.float32),
            pltpu.SemaphoreType.DMA,
        ],
    )
    def k(table_hbm, idx_hbm, out_hbm, idx_v, rows_v, sem):
        wid = lax.axis_index("s") * NC + lax.axis_index("c")
        pltpu.sync_copy(idx_hbm.at[wid], idx_v)
        base = wid * n_per_w

        def body(j, carry):
            pltpu.async_copy(table_hbm.at[idx_v.at[j]], rows_v, sem).wait()
            pltpu.sync_copy(rows_v, out_hbm.at[pl.ds(base + j * CH, CH)])
            return carry

        lax.fori_loop(0, n_ch, body, 0)

    def run(weight, ids_flat):
        idx3 = ids_flat.reshape(NW, n_ch, CH)
        return k(weight, idx3)

    return run


def _mask_body(pad_ref, out_ref):
    r = pl.program_id(1)
    R, S = out_ref.shape[2], out_ref.shape[3]
    rows = r * R + lax.broadcasted_iota(jnp.int32, (R, S), 0)
    cols = lax.broadcasted_iota(jnp.int32, (R, S), 1)
    pad = pad_ref[0, 0, :]
    cond = (cols > rows) | (pad[None, :] == 0)
    out_ref[0, 0] = jnp.where(cond, jnp.int32(INT_MIN), jnp.int32(0))


@functools.lru_cache(maxsize=None)
def _make_mask(Bsz, S):
    R = 256
    return pl.pallas_call(
        _mask_body,
        grid=(Bsz, S // R),
        in_specs=[pl.BlockSpec((1, 1, S), lambda b, r: (b, 0, 0))],
        out_specs=pl.BlockSpec((1, 1, R, S), lambda b, r: (b, 0, r, 0)),
        out_shape=jax.ShapeDtypeStruct((Bsz, 1, S, S), jnp.int32),
    )


def kernel(input_ids, attention_mask, position_ids, labels, weight):
    V, D = weight.shape
    Bsz, S = input_ids.shape
    ids = jnp.clip(input_ids.astype(jnp.int32), 0, V - 1)
    hs = _make_gather(V, D, Bsz * S)(weight, ids.reshape(-1))
    hidden_states = hs.reshape(Bsz, S, D)
    mask = _make_mask(Bsz, S)(attention_mask.astype(jnp.int32).reshape(Bsz, 1, S))
    return (
        hidden_states,
        mask,
        position_ids.astype(jnp.int32),
        jnp.clip(labels.astype(jnp.int32), -100, V - 1),
    )
